# Initial kernel scaffold; baseline (speedup 1.0000x reference)
import jax
import jax.numpy as jnp
from jax.experimental import pallas as pl


def kernel(x, edge_index, edge_attr, query_embedding, W_in, b_in, W_q, b_q, Wl0, bl0, Wr0, g0, be0, Wl1, bl1, Wr1, g1, be1, W_att, b_att):
    raise NotImplementedError("write your pallas kernel here")


# speedup vs baseline: 4.7626x; 4.7626x over previous
import functools

import jax
import jax.numpy as jnp
from jax import lax
from jax.experimental import pallas as pl
from jax.experimental.pallas import tpu as pltpu
from jax.experimental.pallas import tpu_sc as plsc

_N = 10000
_E = 320000
_DH = 128
_CW = 16
_NC = 2
_NS = 16
_NW = _NC * _NS
_EPW = _E // _NW
_CHUNK = 80
_NCHUNK = _EPW // _CHUNK
_RPT = _N // _NS


def _sc_agg_body(with_cnt, h_hbm, src_hbm, dst_hbm, zsum_hbm, zcnt_hbm,
                 ones_hbm, sum_out, cnt_out, acc, cnt, sidx, didx, rows,
                 ones_v, sem):
    cid = lax.axis_index("c")
    sid = lax.axis_index("s")
    wid = sid * _NC + cid
    r0 = sid * _RPT
    pltpu.sync_copy(zsum_hbm.at[pl.ds(r0, _RPT)], acc.at[pl.ds(r0, _RPT)])
    if with_cnt:
        pltpu.sync_copy(zcnt_hbm.at[pl.ds(r0, _RPT)], cnt.at[pl.ds(r0, _RPT)])
        pltpu.sync_copy(ones_hbm, ones_v)
    plsc.subcore_barrier()

    base = wid * _EPW

    def body(c, carry):
        off = pl.multiple_of(base + c * _CHUNK, 8)
        pltpu.sync_copy(src_hbm.at[pl.ds(off, _CHUNK)], sidx)
        pltpu.sync_copy(dst_hbm.at[pl.ds(off, _CHUNK)], didx)
        pltpu.async_copy(h_hbm.at[sidx], rows, sem).wait()
        pltpu.sync_copy(rows, acc.at[didx], add=True)
        if with_cnt:
            pltpu.sync_copy(ones_v, cnt.at[didx], add=True)
        return carry

    lax.fori_loop(0, _NCHUNK, body, 0)
    plsc.subcore_barrier()
    pltpu.sync_copy(acc.at[pl.ds(r0, _RPT)], sum_out.at[cid, pl.ds(r0, _RPT)])
    if with_cnt:
        pltpu.sync_copy(cnt.at[pl.ds(r0, _RPT)],
                        cnt_out.at[cid, pl.ds(r0, _RPT)])


def _sc_aggregate(h, src, dst, with_cnt):
    mesh = plsc.VectorSubcoreMesh(core_axis_name="c", subcore_axis_name="s")
    zsum = jnp.zeros((_N, _DH), jnp.float32)
    zcnt = jnp.zeros((_N, _CW), jnp.float32)
    ones = jnp.ones((_CHUNK, _CW), jnp.float32)
    k = pl.kernel(
        functools.partial(_sc_agg_body, with_cnt),
        out_type=(jax.ShapeDtypeStruct((_NC, _N, _DH), jnp.float32),
                  jax.ShapeDtypeStruct((_NC, _N, _CW), jnp.float32)),
        mesh=mesh,
        scratch_types=[
            pltpu.VMEM_SHARED((_N, _DH), jnp.float32),
            pltpu.VMEM_SHARED((_N, _CW), jnp.float32),
            pltpu.VMEM((_CHUNK,), jnp.int32),
            pltpu.VMEM((_CHUNK,), jnp.int32),
            pltpu.VMEM((_CHUNK, _DH), jnp.float32),
            pltpu.VMEM((_CHUNK, _CW), jnp.float32),
            pltpu.SemaphoreType.DMA,
        ],
        compiler_params=pltpu.CompilerParams(use_tc_tiling_on_sc=False),
    )
    return k(h, src, dst, zsum, zcnt, ones)


def _encode_body(x_ref, wi_ref, bi_ref, q_ref, wq_ref, bq_ref, o_ref):
    q = jnp.dot(q_ref[...], wq_ref[...],
                preferred_element_type=jnp.float32) + bq_ref[...]
    o_ref[...] = jnp.dot(x_ref[...], wi_ref[...],
                         preferred_element_type=jnp.float32) + bi_ref[...] + q


def _combine(h_ref, s0_ref, s1_ref, c0_ref, c1_ref, wl_ref, bl_ref, wr_ref,
             g_ref, be_ref):
    cnt = jnp.maximum((c0_ref[...] + c1_ref[...])[:, :1], 1.0)
    mean = (s0_ref[...] + s1_ref[...]) / cnt
    h = h_ref[...]
    hout = (jnp.dot(mean, wl_ref[...], preferred_element_type=jnp.float32)
            + bl_ref[...]
            + jnp.dot(h, wr_ref[...], preferred_element_type=jnp.float32))
    m = jnp.mean(hout, axis=-1, keepdims=True)
    d = hout - m
    var = jnp.mean(d * d, axis=-1, keepdims=True)
    y = d * lax.rsqrt(var + 1e-5) * g_ref[...] + be_ref[...]
    return h + y


def _layer_body(h_ref, s0_ref, s1_ref, c0_ref, c1_ref, wl_ref, bl_ref,
                wr_ref, g_ref, be_ref, o_ref):
    hn = _combine(h_ref, s0_ref, s1_ref, c0_ref, c1_ref, wl_ref, bl_ref,
                  wr_ref, g_ref, be_ref)
    o_ref[...] = jnp.maximum(hn, 0.0)


def _final_body(h_ref, s0_ref, s1_ref, c0_ref, c1_ref, wl_ref, bl_ref,
                wr_ref, g_ref, be_ref, watt_ref, batt_ref, o_ref, a_ref):
    hn = _combine(h_ref, s0_ref, s1_ref, c0_ref, c1_ref, wl_ref, bl_ref,
                  wr_ref, g_ref, be_ref)
    o_ref[...] = hn
    logits = jnp.dot(hn, watt_ref[...],
                     preferred_element_type=jnp.float32) + batt_ref[...]
    z = logits - jnp.max(logits, axis=0, keepdims=True)
    e = jnp.exp(z)
    a_ref[...] = e / jnp.sum(e, axis=0, keepdims=True)


def kernel(x, edge_index, edge_attr, query_embedding, W_in, b_in, W_q, b_q,
           Wl0, bl0, Wr0, g0, be0, Wl1, bl1, Wr1, g1, be1, W_att, b_att):
    del edge_attr
    src = edge_index[0]
    dst = edge_index[1]
    f32 = jnp.float32
    sds = jax.ShapeDtypeStruct

    h0 = pl.pallas_call(
        _encode_body, out_shape=sds((_N, _DH), f32))(
            x, W_in, b_in.reshape(1, _DH), query_embedding.reshape(1, -1),
            W_q, b_q.reshape(1, _DH))

    sums0, cnts = _sc_aggregate(h0, src, dst, with_cnt=True)
    c0, c1 = cnts[0], cnts[1]

    h1 = pl.pallas_call(
        _layer_body, out_shape=sds((_N, _DH), f32))(
            h0, sums0[0], sums0[1], c0, c1, Wl0, bl0.reshape(1, -1), Wr0,
            g0.reshape(1, -1), be0.reshape(1, -1))

    sums1, _ = _sc_aggregate(h1, src, dst, with_cnt=False)

    h2, attn = pl.pallas_call(
        _final_body, out_shape=(sds((_N, _DH), f32), sds((_N, 1), f32)))(
            h1, sums1[0], sums1[1], c0, c1, Wl1, bl1.reshape(1, -1), Wr1,
            g1.reshape(1, -1), be1.reshape(1, -1), W_att,
            b_att.reshape(1, 1))

    return h2, attn.reshape(-1)


# speedup vs baseline: 8.0700x; 1.6945x over previous
import functools

import jax
import jax.numpy as jnp
from jax import lax
from jax.experimental import pallas as pl
from jax.experimental.pallas import tpu as pltpu
from jax.experimental.pallas import tpu_sc as plsc

_N = 10000
_E = 320000
_DH = 128
_CW = 16
_NC = 2
_NS = 16
_NW = _NC * _NS
_EPW = _E // _NW
_CHUNK = 40
_NCHUNK = _EPW // _CHUNK
_RPT = _N // _NS


def _sc_agg_body(with_cnt, h_hbm, src_hbm, dst_hbm, zsum_hbm, zcnt_hbm,
                 ones_hbm, sum_out, cnt_out, acc, cnt, sidx, didx, rows0,
                 rows1, ones_v, sem0, sem1):
    cid = lax.axis_index("c")
    sid = lax.axis_index("s")
    wid = sid * _NC + cid
    r0 = sid * _RPT
    pltpu.sync_copy(zsum_hbm.at[pl.ds(r0, _RPT)], acc.at[pl.ds(r0, _RPT)])
    if with_cnt:
        pltpu.sync_copy(zcnt_hbm.at[pl.ds(r0, _RPT)], cnt.at[pl.ds(r0, _RPT)])
        pltpu.sync_copy(ones_hbm, ones_v)
    pltpu.sync_copy(src_hbm.at[wid], sidx)
    pltpu.sync_copy(dst_hbm.at[wid], didx)
    pltpu.async_copy(h_hbm.at[sidx.at[0]], rows0, sem0)
    pltpu.async_copy(h_hbm.at[sidx.at[1]], rows1, sem1)
    plsc.subcore_barrier()

    def step(c, buf, sem):
        pltpu.make_async_copy(h_hbm.at[sidx.at[c]], buf, sem).wait()
        pltpu.sync_copy(buf, acc.at[didx.at[c]], add=True)
        if with_cnt:
            pltpu.sync_copy(ones_v, cnt.at[didx.at[c]], add=True)
        nxt = c + 2

        @pl.when(nxt < _NCHUNK)
        def _():
            pltpu.async_copy(h_hbm.at[sidx.at[nxt]], buf, sem)

    @pl.loop(0, _NCHUNK, step=2)
    def _(c):
        step(c, rows0, sem0)
        step(c + 1, rows1, sem1)

    plsc.subcore_barrier()
    pltpu.sync_copy(acc.at[pl.ds(r0, _RPT)], sum_out.at[cid, pl.ds(r0, _RPT)])
    if with_cnt:
        pltpu.sync_copy(cnt.at[pl.ds(r0, _RPT)],
                        cnt_out.at[cid, pl.ds(r0, _RPT)])


def _sc_aggregate(h, src, dst, with_cnt):
    mesh = plsc.VectorSubcoreMesh(core_axis_name="c", subcore_axis_name="s")
    zsum = jnp.zeros((_N, _DH), jnp.float32)
    zcnt = jnp.zeros((_N, _CW), jnp.float32)
    ones = jnp.ones((_CHUNK, _CW), jnp.float32)
    k = pl.kernel(
        functools.partial(_sc_agg_body, with_cnt),
        out_type=(jax.ShapeDtypeStruct((_NC, _N, _DH), jnp.float32),
                  jax.ShapeDtypeStruct((_NC, _N, _CW), jnp.float32)),
        mesh=mesh,
        scratch_types=[
            pltpu.VMEM_SHARED((_N, _DH), jnp.float32),
            pltpu.VMEM_SHARED((_N, _CW), jnp.float32),
            pltpu.VMEM((_NCHUNK, _CHUNK), jnp.int32),
            pltpu.VMEM((_NCHUNK, _CHUNK), jnp.int32),
            pltpu.VMEM((_CHUNK, _DH), jnp.float32),
            pltpu.VMEM((_CHUNK, _DH), jnp.float32),
            pltpu.VMEM((_CHUNK, _CW), jnp.float32),
            pltpu.SemaphoreType.DMA,
            pltpu.SemaphoreType.DMA,
        ],
        compiler_params=pltpu.CompilerParams(use_tc_tiling_on_sc=False),
    )
    src3 = src.reshape(_NW, _NCHUNK, _CHUNK)
    dst3 = dst.reshape(_NW, _NCHUNK, _CHUNK)
    return k(h, src3, dst3, zsum, zcnt, ones)


def _encode_body(x_ref, wi_ref, bi_ref, q_ref, wq_ref, bq_ref, o_ref):
    q = jnp.dot(q_ref[...], wq_ref[...],
                preferred_element_type=jnp.float32) + bq_ref[...]
    o_ref[...] = jnp.dot(x_ref[...], wi_ref[...],
                         preferred_element_type=jnp.float32) + bi_ref[...] + q


def _combine(h_ref, s0_ref, s1_ref, c0_ref, c1_ref, wl_ref, bl_ref, wr_ref,
             g_ref, be_ref):
    cnt = jnp.maximum((c0_ref[...] + c1_ref[...])[:, :1], 1.0)
    mean = (s0_ref[...] + s1_ref[...]) / cnt
    h = h_ref[...]
    hout = (jnp.dot(mean, wl_ref[...], preferred_element_type=jnp.float32)
            + bl_ref[...]
            + jnp.dot(h, wr_ref[...], preferred_element_type=jnp.float32))
    m = jnp.mean(hout, axis=-1, keepdims=True)
    d = hout - m
    var = jnp.mean(d * d, axis=-1, keepdims=True)
    y = d * lax.rsqrt(var + 1e-5) * g_ref[...] + be_ref[...]
    return h + y


def _layer_body(h_ref, s0_ref, s1_ref, c0_ref, c1_ref, wl_ref, bl_ref,
                wr_ref, g_ref, be_ref, o_ref):
    hn = _combine(h_ref, s0_ref, s1_ref, c0_ref, c1_ref, wl_ref, bl_ref,
                  wr_ref, g_ref, be_ref)
    o_ref[...] = jnp.maximum(hn, 0.0)


def _final_body(h_ref, s0_ref, s1_ref, c0_ref, c1_ref, wl_ref, bl_ref,
                wr_ref, g_ref, be_ref, watt_ref, batt_ref, o_ref, a_ref):
    hn = _combine(h_ref, s0_ref, s1_ref, c0_ref, c1_ref, wl_ref, bl_ref,
                  wr_ref, g_ref, be_ref)
    o_ref[...] = hn
    logits = jnp.dot(hn, watt_ref[...],
                     preferred_element_type=jnp.float32) + batt_ref[...]
    z = logits - jnp.max(logits, axis=0, keepdims=True)
    e = jnp.exp(z)
    a_ref[...] = e / jnp.sum(e, axis=0, keepdims=True)


def kernel(x, edge_index, edge_attr, query_embedding, W_in, b_in, W_q, b_q,
           Wl0, bl0, Wr0, g0, be0, Wl1, bl1, Wr1, g1, be1, W_att, b_att):
    del edge_attr
    src = edge_index[0]
    dst = edge_index[1]
    f32 = jnp.float32
    sds = jax.ShapeDtypeStruct

    h0 = pl.pallas_call(
        _encode_body, out_shape=sds((_N, _DH), f32))(
            x, W_in, b_in.reshape(1, _DH), query_embedding.reshape(1, -1),
            W_q, b_q.reshape(1, _DH))

    sums0, cnts = _sc_aggregate(h0, src, dst, with_cnt=True)
    c0, c1 = cnts[0], cnts[1]

    h1 = pl.pallas_call(
        _layer_body, out_shape=sds((_N, _DH), f32))(
            h0, sums0[0], sums0[1], c0, c1, Wl0, bl0.reshape(1, -1), Wr0,
            g0.reshape(1, -1), be0.reshape(1, -1))

    sums1, _ = _sc_aggregate(h1, src, dst, with_cnt=False)

    h2, attn = pl.pallas_call(
        _final_body, out_shape=(sds((_N, _DH), f32), sds((_N, 1), f32)))(
            h1, sums1[0], sums1[1], c0, c1, Wl1, bl1.reshape(1, -1), Wr1,
            g1.reshape(1, -1), be1.reshape(1, -1), W_att,
            b_att.reshape(1, 1))

    return h2, attn.reshape(-1)


# speedup vs baseline: 9.4318x; 1.1688x over previous
import functools

import jax
import jax.numpy as jnp
from jax import lax
from jax.experimental import pallas as pl
from jax.experimental.pallas import tpu as pltpu
from jax.experimental.pallas import tpu_sc as plsc

_N = 10000
_E = 320000
_DH = 128
_CW = 16
_NC = 2
_NS = 16
_NW = _NC * _NS
_EPW = _E // _NW
_CHUNK = 40
_NCHUNK = _EPW // _CHUNK
_RPT = _N // _NS


def _sc_agg_body(with_cnt, h_hbm, src_hbm, dst_hbm, zsum_hbm, zcnt_hbm,
                 ones_hbm, sum_out, cnt_out, acc, cnt, sidx, didx, rows0,
                 rows1, rows2, ones_v, g0, g1, g2, s0, s1, s2):
    cid = lax.axis_index("c")
    sid = lax.axis_index("s")
    wid = sid * _NC + cid
    r0 = sid * _RPT
    rows = (rows0, rows1, rows2)
    gsem = (g0, g1, g2)
    ssem = (s0, s1, s2)
    pltpu.sync_copy(zsum_hbm.at[pl.ds(r0, _RPT)], acc.at[pl.ds(r0, _RPT)])
    if with_cnt:
        pltpu.sync_copy(zcnt_hbm.at[pl.ds(r0, _RPT)], cnt.at[pl.ds(r0, _RPT)])
        pltpu.sync_copy(ones_hbm, ones_v)
    pltpu.sync_copy(src_hbm.at[wid], sidx)
    pltpu.sync_copy(dst_hbm.at[wid], didx)
    pltpu.async_copy(h_hbm.at[sidx.at[0]], rows0, g0)
    pltpu.async_copy(h_hbm.at[sidx.at[1]], rows1, g1)
    plsc.subcore_barrier()

    def wait_scatters(c, buf):
        pltpu.make_async_copy(rows[buf], acc.at[didx.at[c]],
                              ssem[buf]).wait()
        if with_cnt:
            pltpu.make_async_copy(ones_v, cnt.at[didx.at[c]],
                                  ssem[buf]).wait()

    def step(c, buf):
        pltpu.make_async_copy(h_hbm.at[sidx.at[c]], rows[buf],
                              gsem[buf]).wait()
        pltpu.make_async_copy(rows[buf], acc.at[didx.at[c]],
                              ssem[buf]).start(add=True)
        if with_cnt:
            pltpu.make_async_copy(ones_v, cnt.at[didx.at[c]],
                                  ssem[buf]).start(add=True)
        pbuf = (buf - 1) % 3
        nxt = c + 2
        if isinstance(c, int):
            if c >= 1:
                wait_scatters(c, pbuf)
            if nxt < _NCHUNK:
                pltpu.make_async_copy(h_hbm.at[sidx.at[nxt]], rows[pbuf],
                                      gsem[pbuf]).start()
        else:
            @pl.when(c >= 1)
            def _():
                wait_scatters(c, pbuf)

            @pl.when(nxt < _NCHUNK)
            def _():
                pltpu.make_async_copy(h_hbm.at[sidx.at[nxt]], rows[pbuf],
                                      gsem[pbuf]).start()

    @pl.loop(0, _NCHUNK - 1, step=3)
    def _(c):
        step(c, 0)
        step(c + 1, 1)
        step(c + 2, 2)

    step(_NCHUNK - 1, 0)
    wait_scatters(0, 0)

    plsc.subcore_barrier()
    pltpu.sync_copy(acc.at[pl.ds(r0, _RPT)], sum_out.at[cid, pl.ds(r0, _RPT)])
    if with_cnt:
        pltpu.sync_copy(cnt.at[pl.ds(r0, _RPT)],
                        cnt_out.at[cid, pl.ds(r0, _RPT)])


def _sc_aggregate(h, src, dst, with_cnt):
    mesh = plsc.VectorSubcoreMesh(core_axis_name="c", subcore_axis_name="s")
    zsum = jnp.zeros((_N, _DH), jnp.float32)
    zcnt = jnp.zeros((_N, _CW), jnp.float32)
    ones = jnp.ones((_CHUNK, _CW), jnp.float32)
    k = pl.kernel(
        functools.partial(_sc_agg_body, with_cnt),
        out_type=(jax.ShapeDtypeStruct((_NC, _N, _DH), jnp.float32),
                  jax.ShapeDtypeStruct((_NC, _N, _CW), jnp.float32)),
        mesh=mesh,
        scratch_types=[
            pltpu.VMEM_SHARED((_N, _DH), jnp.float32),
            pltpu.VMEM_SHARED((_N, _CW), jnp.float32),
            pltpu.VMEM((_NCHUNK, _CHUNK), jnp.int32),
            pltpu.VMEM((_NCHUNK, _CHUNK), jnp.int32),
            pltpu.VMEM((_CHUNK, _DH), jnp.float32),
            pltpu.VMEM((_CHUNK, _DH), jnp.float32),
            pltpu.VMEM((_CHUNK, _DH), jnp.float32),
            pltpu.VMEM((_CHUNK, _CW), jnp.float32),
            pltpu.SemaphoreType.DMA,
            pltpu.SemaphoreType.DMA,
            pltpu.SemaphoreType.DMA,
            pltpu.SemaphoreType.DMA,
            pltpu.SemaphoreType.DMA,
            pltpu.SemaphoreType.DMA,
        ],
        compiler_params=pltpu.CompilerParams(use_tc_tiling_on_sc=False),
    )
    src3 = src.reshape(_NW, _NCHUNK, _CHUNK)
    dst3 = dst.reshape(_NW, _NCHUNK, _CHUNK)
    return k(h, src3, dst3, zsum, zcnt, ones)


def _encode_body(x_ref, wi_ref, bi_ref, q_ref, wq_ref, bq_ref, o_ref):
    q = jnp.dot(q_ref[...], wq_ref[...],
                preferred_element_type=jnp.float32) + bq_ref[...]
    o_ref[...] = jnp.dot(x_ref[...], wi_ref[...],
                         preferred_element_type=jnp.float32) + bi_ref[...] + q


def _combine(h_ref, s0_ref, s1_ref, c0_ref, c1_ref, wl_ref, bl_ref, wr_ref,
             g_ref, be_ref):
    cnt = jnp.maximum((c0_ref[...] + c1_ref[...])[:, :1], 1.0)
    mean = (s0_ref[...] + s1_ref[...]) / cnt
    h = h_ref[...]
    hout = (jnp.dot(mean, wl_ref[...], preferred_element_type=jnp.float32)
            + bl_ref[...]
            + jnp.dot(h, wr_ref[...], preferred_element_type=jnp.float32))
    m = jnp.mean(hout, axis=-1, keepdims=True)
    d = hout - m
    var = jnp.mean(d * d, axis=-1, keepdims=True)
    y = d * lax.rsqrt(var + 1e-5) * g_ref[...] + be_ref[...]
    return h + y


def _layer_body(h_ref, s0_ref, s1_ref, c0_ref, c1_ref, wl_ref, bl_ref,
                wr_ref, g_ref, be_ref, o_ref):
    hn = _combine(h_ref, s0_ref, s1_ref, c0_ref, c1_ref, wl_ref, bl_ref,
                  wr_ref, g_ref, be_ref)
    o_ref[...] = jnp.maximum(hn, 0.0)


def _final_body(h_ref, s0_ref, s1_ref, c0_ref, c1_ref, wl_ref, bl_ref,
                wr_ref, g_ref, be_ref, watt_ref, batt_ref, o_ref, a_ref):
    hn = _combine(h_ref, s0_ref, s1_ref, c0_ref, c1_ref, wl_ref, bl_ref,
                  wr_ref, g_ref, be_ref)
    o_ref[...] = hn
    logits = jnp.dot(hn, watt_ref[...],
                     preferred_element_type=jnp.float32) + batt_ref[...]
    z = logits - jnp.max(logits, axis=0, keepdims=True)
    e = jnp.exp(z)
    a_ref[...] = e / jnp.sum(e, axis=0, keepdims=True)


def kernel(x, edge_index, edge_attr, query_embedding, W_in, b_in, W_q, b_q,
           Wl0, bl0, Wr0, g0, be0, Wl1, bl1, Wr1, g1, be1, W_att, b_att):
    del edge_attr
    src = edge_index[0]
    dst = edge_index[1]
    f32 = jnp.float32
    sds = jax.ShapeDtypeStruct

    h0 = pl.pallas_call(
        _encode_body, out_shape=sds((_N, _DH), f32))(
            x, W_in, b_in.reshape(1, _DH), query_embedding.reshape(1, -1),
            W_q, b_q.reshape(1, _DH))

    sums0, cnts = _sc_aggregate(h0, src, dst, with_cnt=True)
    c0, c1 = cnts[0], cnts[1]

    h1 = pl.pallas_call(
        _layer_body, out_shape=sds((_N, _DH), f32))(
            h0, sums0[0], sums0[1], c0, c1, Wl0, bl0.reshape(1, -1), Wr0,
            g0.reshape(1, -1), be0.reshape(1, -1))

    sums1, _ = _sc_aggregate(h1, src, dst, with_cnt=False)

    h2, attn = pl.pallas_call(
        _final_body, out_shape=(sds((_N, _DH), f32), sds((_N, 1), f32)))(
            h1, sums1[0], sums1[1], c0, c1, Wl1, bl1.reshape(1, -1), Wr1,
            g1.reshape(1, -1), be1.reshape(1, -1), W_att,
            b_att.reshape(1, 1))

    return h2, attn.reshape(-1)
